# Initial kernel scaffold; baseline (speedup 1.0000x reference)
import functools

import jax
import jax.numpy as jnp
from jax import lax
from jax.experimental import pallas as pl
from jax.experimental.pallas import tpu as pltpu
from jax.experimental.pallas import tpu_sc as plsc

N_NODES_C = 10000
N_EDGES_C = 320000
D_C = 128

NUM_CORES = 2
NUM_SUBCORES = 16
NW = NUM_CORES * NUM_SUBCORES
E_PER_W = N_EDGES_C // NW
CHUNK = 80
NCHUNK = E_PER_W // CHUNK
ROWS_PER_TILE = N_NODES_C // NUM_SUBCORES
LANES = 16
GROUPS = D_C // LANES


def _sc_segment_sum(ego, src_r, dst_r, ev_r):
  mesh = plsc.VectorSubcoreMesh(core_axis_name="c", subcore_axis_name="s")

  @functools.partial(
      pl.kernel,
      out_type=jax.ShapeDtypeStruct((NUM_CORES, N_NODES_C, D_C), jnp.float32),
      mesh=mesh,
      scratch_types=[
          pltpu.VMEM((NCHUNK, CHUNK), jnp.int32),
          pltpu.VMEM((NCHUNK, CHUNK), jnp.int32),
          pltpu.VMEM((NCHUNK, CHUNK), jnp.float32),
          pltpu.VMEM((CHUNK, D_C), jnp.float32),
          pltpu.VMEM_SHARED((N_NODES_C, D_C), jnp.float32),
          pltpu.SemaphoreType.DMA,
      ],
  )
  def k(ego_hbm, src_hbm, dst_hbm, ev_hbm, out_hbm,
        src_v, dst_v, ev_v, rows_v, acc, sem):
    c = lax.axis_index("c")
    s = lax.axis_index("s")
    wid = c * NUM_SUBCORES + s

    pltpu.sync_copy(src_hbm.at[wid], src_v)
    pltpu.sync_copy(dst_hbm.at[wid], dst_v)
    pltpu.sync_copy(ev_hbm.at[wid], ev_v)

    zero = jnp.zeros((LANES,), jnp.float32)

    def zrow(i, _):
      for g in range(GROUPS):
        rows_v[i, pl.ds(g * LANES, LANES)] = zero
      return 0

    lax.fori_loop(0, CHUNK, zrow, 0)

    stripe = s * ROWS_PER_TILE
    n_full = ROWS_PER_TILE // CHUNK
    rem = ROWS_PER_TILE - n_full * CHUNK

    def zcopy(kk, _):
      pltpu.sync_copy(rows_v, acc.at[pl.ds(stripe + kk * CHUNK, CHUNK)])
      return 0

    lax.fori_loop(0, n_full, zcopy, 0)
    pltpu.sync_copy(rows_v.at[pl.ds(0, rem)],
                    acc.at[pl.ds(stripe + n_full * CHUNK, rem)])
    plsc.subcore_barrier()

    def chunk_body(j, _):
      pltpu.async_copy(ego_hbm.at[src_v.at[j]], rows_v, sem).wait()

      def row_body(i, _):
        evv = plsc.load_gather(
            ev_v, [jnp.full((LANES,), j, jnp.int32),
                   jnp.full((LANES,), i, jnp.int32)])
        for g in range(GROUPS):
          sl = pl.ds(g * LANES, LANES)
          rows_v[i, sl] = rows_v[i, sl] * evv
        return 0

      lax.fori_loop(0, CHUNK, row_body, 0)
      pltpu.sync_copy(rows_v, acc.at[dst_v.at[j]], add=True)
      return 0

    lax.fori_loop(0, NCHUNK, chunk_body, 0)
    plsc.subcore_barrier()

    pltpu.sync_copy(acc.at[pl.ds(stripe, ROWS_PER_TILE)],
                    out_hbm.at[c, pl.ds(stripe, ROWS_PER_TILE)])

  return k(ego, src_r, dst_r, ev_r)


def _tc_linear(ego, p0, p1, W, b2d):
  R = 1000
  grid = (N_NODES_C // R,)

  def body(ego_ref, p0_ref, p1_ref, w_ref, b_ref, out_ref):
    x = ego_ref[...] + p0_ref[...] + p1_ref[...]
    y = lax.dot_general(x, w_ref[...], (((1,), (1,)), ((), ())),
                        preferred_element_type=jnp.float32)
    y = y + b_ref[...]
    out_ref[...] = jnp.where(y >= 0, y, 0.01 * y)

  return pl.pallas_call(
      body,
      grid=grid,
      in_specs=[
          pl.BlockSpec((R, D_C), lambda i: (i, 0)),
          pl.BlockSpec((R, D_C), lambda i: (i, 0)),
          pl.BlockSpec((R, D_C), lambda i: (i, 0)),
          pl.BlockSpec((D_C, D_C), lambda i: (0, 0)),
          pl.BlockSpec((1, D_C), lambda i: (0, 0)),
      ],
      out_specs=pl.BlockSpec((R, D_C), lambda i: (i, 0)),
      out_shape=jax.ShapeDtypeStruct((N_NODES_C, D_C), jnp.float32),
  )(ego, p0, p1, W, b2d)


@jax.jit
def kernel(edge_index, edge_values, ego_embeddings, W, b):
  src_r = edge_index[0].reshape(NW, NCHUNK, CHUNK)
  dst_r = edge_index[1].reshape(NW, NCHUNK, CHUNK)
  ev_r = edge_values.reshape(NW, NCHUNK, CHUNK)
  partials = _sc_segment_sum(ego_embeddings, src_r, dst_r, ev_r)
  b2d = b.reshape(1, D_C)
  return _tc_linear(ego_embeddings, partials[0], partials[1], W, b2d)


# speedup vs baseline: 4.6619x; 4.6619x over previous
import functools

import jax
import jax.numpy as jnp
from jax import lax
from jax.experimental import pallas as pl
from jax.experimental.pallas import tpu as pltpu
from jax.experimental.pallas import tpu_sc as plsc

N_NODES_C = 10000
N_EDGES_C = 320000
D_C = 128

NUM_CORES = 2
NUM_SUBCORES = 16
NW = NUM_CORES * NUM_SUBCORES
E_PER_W = N_EDGES_C // NW
CHUNK = 80
NCHUNK = E_PER_W // CHUNK
STRIPE = 624
TAIL = N_NODES_C - NUM_SUBCORES * STRIPE
LANES = 16
GROUPS = D_C // LANES


def _sc_segment_sum(ego, packed, ev_r):
  mesh = plsc.VectorSubcoreMesh(core_axis_name="c", subcore_axis_name="s")

  @functools.partial(
      pl.kernel,
      out_type=jax.ShapeDtypeStruct((NUM_CORES, N_NODES_C, D_C), jnp.float32),
      mesh=mesh,
      scratch_types=[
          pltpu.VMEM((2, CHUNK), jnp.int32),
          pltpu.VMEM((CHUNK,), jnp.float32),
          pltpu.VMEM((CHUNK, D_C), jnp.float32),
          pltpu.VMEM_SHARED((N_NODES_C, D_C), jnp.float32),
          pltpu.SemaphoreType.DMA,
      ],
  )
  def k(ego_hbm, packed_hbm, ev_hbm, out_hbm, pack_v, ev_v, rows_v, acc, sem):
    c = lax.axis_index("c")
    s = lax.axis_index("s")
    wid = c * NUM_SUBCORES + s

    zero = jnp.zeros((LANES,), jnp.float32)

    def zrow(i, _):
      for g in range(GROUPS):
        rows_v[i, pl.ds(g * LANES, LANES)] = zero
      return 0

    lax.fori_loop(0, CHUNK, zrow, 0)

    stripe = s * STRIPE
    n_full = STRIPE // CHUNK
    rem = STRIPE - n_full * CHUNK

    def zcopy(kk, _):
      pltpu.sync_copy(rows_v, acc.at[pl.ds(stripe + kk * CHUNK, CHUNK)])
      return 0

    lax.fori_loop(0, n_full, zcopy, 0)
    pltpu.sync_copy(rows_v.at[pl.ds(0, rem)],
                    acc.at[pl.ds(stripe + n_full * CHUNK, rem)])

    @pl.when(s == NUM_SUBCORES - 1)
    def _zero_tail():
      pltpu.sync_copy(rows_v.at[pl.ds(0, TAIL)],
                      acc.at[pl.ds(NUM_SUBCORES * STRIPE, TAIL)])

    plsc.subcore_barrier()

    def chunk_body(j, _):
      pltpu.sync_copy(packed_hbm.at[wid, j], pack_v)
      pltpu.sync_copy(ev_hbm.at[wid, j], ev_v)
      pltpu.async_copy(ego_hbm.at[pack_v.at[0]], rows_v, sem).wait()

      def group_body(ib, _):
        evs = ev_v[pl.ds(ib * LANES, LANES)]
        for l in range(LANES):
          evv = jnp.full((LANES,), evs[l], jnp.float32)
          row = ib * LANES + l
          for g in range(GROUPS):
            sl = pl.ds(g * LANES, LANES)
            rows_v[row, sl] = rows_v[row, sl] * evv
        return 0

      lax.fori_loop(0, CHUNK // LANES, group_body, 0)
      pltpu.sync_copy(rows_v, acc.at[pack_v.at[1]], add=True)
      return 0

    lax.fori_loop(0, NCHUNK, chunk_body, 0)
    plsc.subcore_barrier()

    pltpu.sync_copy(acc.at[pl.ds(stripe, STRIPE)],
                    out_hbm.at[c, pl.ds(stripe, STRIPE)])

    @pl.when(s == NUM_SUBCORES - 1)
    def _write_tail():
      pltpu.sync_copy(acc.at[pl.ds(NUM_SUBCORES * STRIPE, TAIL)],
                      out_hbm.at[c, pl.ds(NUM_SUBCORES * STRIPE, TAIL)])

  return k(ego, packed, ev_r)


def _tc_linear(ego, p0, p1, W, b2d):
  R = 1000
  grid = (N_NODES_C // R,)

  def body(ego_ref, p0_ref, p1_ref, w_ref, b_ref, out_ref):
    x = ego_ref[...] + p0_ref[...] + p1_ref[...]
    y = lax.dot_general(x, w_ref[...], (((1,), (1,)), ((), ())),
                        preferred_element_type=jnp.float32)
    y = y + b_ref[...]
    out_ref[...] = jnp.where(y >= 0, y, 0.01 * y)

  return pl.pallas_call(
      body,
      grid=grid,
      in_specs=[
          pl.BlockSpec((R, D_C), lambda i: (i, 0)),
          pl.BlockSpec((R, D_C), lambda i: (i, 0)),
          pl.BlockSpec((R, D_C), lambda i: (i, 0)),
          pl.BlockSpec((D_C, D_C), lambda i: (0, 0)),
          pl.BlockSpec((1, D_C), lambda i: (0, 0)),
      ],
      out_specs=pl.BlockSpec((R, D_C), lambda i: (i, 0)),
      out_shape=jax.ShapeDtypeStruct((N_NODES_C, D_C), jnp.float32),
  )(ego, p0, p1, W, b2d)


@jax.jit
def kernel(edge_index, edge_values, ego_embeddings, W, b):
  src_r = edge_index[0].reshape(NW, NCHUNK, CHUNK)
  dst_r = edge_index[1].reshape(NW, NCHUNK, CHUNK)
  ev_r = edge_values.reshape(NW, NCHUNK, CHUNK)
  packed = jnp.stack([src_r, dst_r], axis=2)
  partials = _sc_segment_sum(ego_embeddings, packed, ev_r)
  b2d = b.reshape(1, D_C)
  return _tc_linear(ego_embeddings, partials[0], partials[1], W, b2d)
